# Initial kernel scaffold; baseline (speedup 1.0000x reference)
import jax
import jax.numpy as jnp
from jax.experimental import pallas as pl


def kernel(x, edge_index, params):
    raise NotImplementedError("write your pallas kernel here")


# speedup vs baseline: 1.4979x; 1.4979x over previous
import functools

import jax
import jax.numpy as jnp
from jax import lax
from jax.experimental import pallas as pl
from jax.experimental.pallas import tpu as pltpu
from jax.experimental.pallas import tpu_sc as plsc

N_NODES = 10000
N_EDGES = 320000
NC, NS, LANES = 2, 16, 16
NW = NC * NS
NPT = 320
NPAD = NW * NPT
EPT = N_EDGES // NW
GB = 80
CH = 4000
FLUSH = 128

_mesh = plsc.VectorSubcoreMesh(core_axis_name="c", subcore_axis_name="s")


def _wid():
    return lax.axis_index("s") * NC + lax.axis_index("c")


def _relu(v):
    return jnp.maximum(v, 0.0)


def _dot(a, b):
    return jnp.dot(a, b, preferred_element_type=jnp.float32)


def _stage0(x, w1, b1):
    d = x.shape[1]

    def body(x_ref, w_ref, b_ref, pq_ref):
        w = w_ref[...]
        a = w[0:d] - w[2 * d:3 * d]
        bb = w[d:2 * d] + w[2 * d:3 * d]
        xx = x_ref[...]
        pq_ref[...] = jnp.concatenate(
            [_dot(xx, a) + b_ref[...], _dot(xx, bb)], axis=1)

    return pl.pallas_call(
        body,
        out_shape=jax.ShapeDtypeStruct((N_NODES, 128), jnp.float32),
    )(x, w1, b1.reshape(1, 64))


def _edge_mlp(ee1, ee2, w2, b2, w3, b3):
    BE = 2000
    grid = N_EDGES // BE

    def body(e1_ref, e2_ref, w2_ref, b2_ref, w3_ref, b3_ref, out_ref):
        h = _relu(e1_ref[...][:, 0:64] + e2_ref[...][:, 64:128])
        h = _relu(_dot(h, w2_ref[...]) + b2_ref[...])
        m = _dot(h, w3_ref[...]) + b3_ref[...]
        out_ref[...] = jnp.concatenate([m, jnp.zeros_like(m)], axis=1)

    full = lambda i: (0, 0)
    return pl.pallas_call(
        body,
        grid=(grid,),
        in_specs=[pl.BlockSpec((BE, 128), lambda i: (i, 0)),
                  pl.BlockSpec((BE, 128), lambda i: (i, 0)),
                  pl.BlockSpec((64, 64), full),
                  pl.BlockSpec((1, 64), full),
                  pl.BlockSpec((64, 64), full),
                  pl.BlockSpec((1, 64), full)],
        out_specs=pl.BlockSpec((BE, 128), lambda i: (i, 0)),
        out_shape=jax.ShapeDtypeStruct((N_EDGES, 128), jnp.float32),
    )(ee1, ee2, w2, b2.reshape(1, 64), w3, b3.reshape(1, 64))


def _stage1(aggr, x, gamma, phi2_w, phi2_b):
    (g1w, g1b), (g2w, g2b), (g3w, g3b) = gamma
    dx = x.shape[1]

    def body(a_ref, x_ref, g1w_ref, g1b_ref, g2w_ref, g2b_ref, g3w_ref,
             g3b_ref, pw_ref, pb_ref, h_ref, pq_ref):
        g1 = g1w_ref[...]
        t = _relu(_dot(a_ref[...], g1[0:64]) + _dot(x_ref[...], g1[64:64 + dx])
                  + g1b_ref[...])
        t = _relu(_dot(t, g2w_ref[...]) + g2b_ref[...])
        h = _relu(_dot(t, g3w_ref[...]) + g3b_ref[...])
        h_ref[...] = h
        pw = pw_ref[...]
        a2 = pw[0:64] - pw[128:192]
        b2 = pw[64:128] + pw[128:192]
        pq_ref[...] = jnp.concatenate(
            [_dot(h, a2) + pb_ref[...], _dot(h, b2)], axis=1)

    return pl.pallas_call(
        body,
        out_shape=[jax.ShapeDtypeStruct((N_NODES, 64), jnp.float32),
                   jax.ShapeDtypeStruct((N_NODES, 128), jnp.float32)],
    )(aggr, x, g1w, g1b.reshape(1, 64), g2w, g2b.reshape(1, 64),
      g3w, g3b.reshape(1, 64), phi2_w, phi2_b.reshape(1, 64))


def _stage2(aggr, h1, gamma, head):
    (g1w, g1b), (g2w, g2b), (g3w, g3b) = gamma
    (h1w, h1b), (h2w, h2b), (h3w, h3b) = head
    nout = h3w.shape[1]

    def body(a_ref, x_ref, g1w_ref, g1b_ref, g2w_ref, g2b_ref, g3w_ref,
             g3b_ref, h1w_ref, h1b_ref, h2w_ref, h2b_ref, h3w_ref, h3b_ref,
             out_ref):
        g1 = g1w_ref[...]
        t = _relu(_dot(a_ref[...], g1[0:64]) + _dot(x_ref[...], g1[64:128])
                  + g1b_ref[...])
        t = _relu(_dot(t, g2w_ref[...]) + g2b_ref[...])
        g = _dot(t, g3w_ref[...]) + g3b_ref[...]
        v = _relu(_dot(g, h1w_ref[...]) + h1b_ref[...])
        v = _relu(_dot(v, h2w_ref[...]) + h2b_ref[...])
        out_ref[...] = _dot(v, h3w_ref[...]) + h3b_ref[...]

    return pl.pallas_call(
        body,
        out_shape=jax.ShapeDtypeStruct((N_NODES, nout), jnp.float32),
    )(aggr, h1, g1w, g1b.reshape(1, 64), g2w, g2b.reshape(1, 64),
      g3w, g3b.reshape(1, 64), h1w, h1b.reshape(1, 64),
      h2w, h2b.reshape(1, 64), h3w, h3b.reshape(1, nout))


@functools.partial(
    pl.kernel,
    out_type=[jax.ShapeDtypeStruct((N_EDGES, 128), jnp.float32),
              jax.ShapeDtypeStruct((N_EDGES, 128), jnp.float32)],
    mesh=_mesh,
    compiler_params=pltpu.CompilerParams(needs_layout_passes=False),
    scratch_types=[
        pltpu.VMEM((EPT,), jnp.int32),
        pltpu.VMEM((EPT,), jnp.int32),
        pltpu.VMEM((GB, 128), jnp.float32),
        pltpu.VMEM((GB, 128), jnp.float32),
        pltpu.SemaphoreType.DMA,
        pltpu.SemaphoreType.DMA,
    ],
)
def _sc_gather(pq_hbm, dst_hbm, src_hbm, e1_hbm, e2_hbm,
               dbuf, sbuf, buf_a, buf_b, sem_a, sem_b):
    base = _wid() * EPT
    pltpu.sync_copy(dst_hbm.at[pl.ds(base, EPT)], dbuf)
    pltpu.sync_copy(src_hbm.at[pl.ds(base, EPT)], sbuf)

    def grp(g, carry):
        off = g * GB
        ca = pltpu.async_copy(pq_hbm.at[dbuf.at[pl.ds(off, GB)]], buf_a, sem_a)
        cb = pltpu.async_copy(pq_hbm.at[sbuf.at[pl.ds(off, GB)]], buf_b, sem_b)
        ca.wait()
        cb.wait()
        pltpu.sync_copy(buf_a, e1_hbm.at[pl.ds(base + off, GB)])
        pltpu.sync_copy(buf_b, e2_hbm.at[pl.ds(base + off, GB)])
        return carry

    lax.fori_loop(0, EPT // GB, grp, 0)


@functools.partial(
    pl.kernel,
    out_type=jax.ShapeDtypeStruct((NPAD, 64), jnp.float32),
    mesh=_mesh,
    compiler_params=pltpu.CompilerParams(needs_layout_passes=False),
    scratch_types=[
        pltpu.VMEM((CH,), jnp.int32),
        pltpu.VMEM((FLUSH + 32,), jnp.int32),
        pltpu.VMEM((FLUSH + 32,), jnp.int32),
        pltpu.VMEM((FLUSH,), jnp.int32),
        pltpu.VMEM((FLUSH, 128), jnp.float32),
        pltpu.VMEM((NPT + 1, 64), jnp.float32),
        pltpu.SemaphoreType.DMA,
    ],
)
def _sc_segmax(m_hbm, dst_hbm, out_hbm, dbuf, pend_e, pend_d, gidx, rows,
               aggr, sem):
    wid = _wid()
    lo = wid * NPT
    hi = lo + NPT
    neg = jnp.full((LANES,), -jnp.inf, jnp.float32)
    lane = lax.iota(jnp.int32, LANES)

    def ini(t, carry):
        for f in range(4):
            aggr[t, pl.ds(f * 16, 16)] = neg
        return carry

    lax.fori_loop(0, NPT + 1, ini, 0)

    def flush():
        for k in range(FLUSH // LANES):
            gidx[pl.ds(k * LANES, LANES)] = pend_e[pl.ds(k * LANES, LANES)]
        pltpu.async_copy(m_hbm.at[gidx], rows, sem).wait()

        def upd(j, carry):
            dl = pend_d[pl.ds(j, 16)][0]
            for f in range(4):
                sl = pl.ds(f * 16, 16)
                aggr[dl, sl] = jnp.maximum(aggr[dl, sl], rows[j, sl])
            return carry

        lax.fori_loop(0, FLUSH, upd, 0)

    def chunk(c, pcnt):
        pltpu.sync_copy(dst_hbm.at[pl.ds(c * CH, CH)], dbuf)

        def step(i, pcnt):
            d = dbuf[pl.ds(i * 16, 16)]
            m = (d >= lo) & (d < hi)
            eg = c * CH + i * 16 + lane
            pos = pcnt + plsc.cumsum(m.astype(jnp.int32)) - 1
            plsc.store_scatter(pend_e, [pos], eg, mask=m)
            plsc.store_scatter(pend_d, [pos], d - lo, mask=m)
            npc = pcnt + jnp.max(plsc.all_reduce_population_count(m))

            @pl.when(npc >= FLUSH)
            def _():
                flush()
                rem_e = pend_e[pl.ds(FLUSH, 16)]
                rem_d = pend_d[pl.ds(FLUSH, 16)]
                pend_e[pl.ds(0, 16)] = rem_e
                pend_d[pl.ds(0, 16)] = rem_d

            return jnp.where(npc >= FLUSH, npc - FLUSH, npc)

        return lax.fori_loop(0, CH // 16, step, pcnt)

    pcnt = lax.fori_loop(0, N_EDGES // CH, chunk, 0)

    for g in range(FLUSH // LANES):
        sl = pl.ds(g * LANES, LANES)
        valid = (g * LANES + lane) < pcnt
        pend_e[sl] = jnp.where(valid, pend_e[sl], 0)
        pend_d[sl] = jnp.where(valid, pend_d[sl], NPT)
    flush()

    def fin(t, carry):
        for f in range(4):
            sl = pl.ds(f * 16, 16)
            v = aggr[t, sl]
            aggr[t, sl] = jnp.where(v == -jnp.inf, 0.0, v)
        return carry

    lax.fori_loop(0, NPT, fin, 0)
    pltpu.sync_copy(aggr.at[pl.ds(0, NPT)], out_hbm.at[pl.ds(lo, NPT)])


def kernel(x, edge_index, params):
    src = edge_index[0]
    dst = edge_index[1]

    phi1, gamma1 = params["phi1"], params["gamma1"]
    phi2, gamma2 = params["phi2"], params["gamma2"]
    head = params["head"]

    pq1 = _stage0(x, phi1[0][0], phi1[0][1])
    ee1, ee2 = _sc_gather(pq1, dst, src)
    m1 = _edge_mlp(ee1, ee2, phi1[1][0], phi1[1][1], phi1[2][0], phi1[2][1])
    aggr1 = _sc_segmax(m1, dst)
    h1, pq2 = _stage1(aggr1[:N_NODES], x, gamma1, phi2[0][0], phi2[0][1])

    f1, f2 = _sc_gather(pq2, dst, src)
    m2 = _edge_mlp(f1, f2, phi2[1][0], phi2[1][1], phi2[2][0], phi2[2][1])
    aggr2 = _sc_segmax(m2, dst)

    out = _stage2(aggr2[:N_NODES], h1, gamma2, head)
    return out.reshape((-1, 100, 100))
